# baseline (speedup 1.0000x reference)
import functools
import jax
import jax.numpy as jnp
from jax import lax
from jax.experimental import pallas as pl
from jax.experimental.pallas import tpu as pltpu
from jax.experimental.pallas import tpu_sc as plsc

F32 = jnp.float32
BF16 = jnp.bfloat16
CHUNK = 88
NUM_WORKERS = 32


def _pads(n_nodes, n_edges):
    npad = ((n_nodes + 1 + 127) // 128) * 128
    epad = ((n_edges + NUM_WORKERS * CHUNK - 1) // (NUM_WORKERS * CHUNK)) * (
        NUM_WORKERS * CHUNK)
    return npad, epad


def _zero_rows(buf, ncols16):
    def body(i, _):
        for k in range(ncols16):
            buf[i, pl.ds(k * 16, 16)] = jnp.zeros((16,), F32)
        return 0
    lax.fori_loop(0, CHUNK, body, 0)


def _zero_acc_slice(zbuf, acc_sh, sid, rows_per_tile):
    base = sid * rows_per_tile
    full, rem = rows_per_tile // CHUNK, rows_per_tile % CHUNK
    for r in range(full):
        pltpu.sync_copy(zbuf, acc_sh.at[pl.ds(base + r * CHUNK, CHUNK)])
    if rem:
        pltpu.sync_copy(zbuf.at[pl.ds(0, rem)],
                        acc_sh.at[pl.ds(base + full * CHUNK, rem)])


def _deg_pass(sdidx, npad):
    nchunk = sdidx.shape[0]
    per_tile = nchunk // NUM_WORKERS
    rows_per_tile = npad // 16
    mesh = plsc.VectorSubcoreMesh(core_axis_name="c", subcore_axis_name="s")

    @functools.partial(
        pl.kernel,
        out_type=jax.ShapeDtypeStruct((2, npad, 16), F32),
        mesh=mesh,
        scratch_types=[
            pltpu.VMEM((2, 2, CHUNK), jnp.int32),
            pltpu.VMEM((CHUNK, 16), F32),
            pltpu.VMEM_SHARED((npad, 16), F32),
            pltpu.SemaphoreType.DMA((2,)),
        ],
    )
    def k(sdidx_hbm, out_hbm, idx_v, val_v, acc_sh, isem):
        cid = lax.axis_index("c")
        sid = lax.axis_index("s")
        wid = cid * 16 + sid
        base_g = wid * per_tile
        _zero_rows(val_v, 1)
        _zero_acc_slice(val_v, acc_sh, sid, rows_per_tile)
        plsc.subcore_barrier()

        def fill_ones(i, _):
            val_v[i] = jnp.ones((16,), F32)
            return 0
        lax.fori_loop(0, CHUNK, fill_ones, 0)

        def body(t, _):
            b = lax.rem(t, 2)
            b1 = 1 - b

            @pl.when(t < per_tile)
            def _prefetch():
                pltpu.async_copy(sdidx_hbm.at[base_g + t], idx_v.at[b],
                                 isem.at[b])

            @pl.when(t > 0)
            def _process():
                pltpu.make_async_copy(sdidx_hbm.at[base_g + t - 1],
                                      idx_v.at[b1], isem.at[b1]).wait()
                pltpu.sync_copy(val_v, acc_sh.at[idx_v.at[b1, 0]], add=True)
            return 0
        lax.fori_loop(0, per_tile + 1, body, 0)
        plsc.subcore_barrier()
        pltpu.sync_copy(acc_sh.at[pl.ds(sid * rows_per_tile, rows_per_tile)],
                        out_hbm.at[cid].at[pl.ds(sid * rows_per_tile, rows_per_tile)])

    return k(sdidx)


def _sval_pass(dinv128, sdidx):
    nchunk = sdidx.shape[0]
    epad = nchunk * CHUNK
    per_tile = nchunk // NUM_WORKERS
    mesh = plsc.VectorSubcoreMesh(core_axis_name="c", subcore_axis_name="s")

    @functools.partial(
        pl.kernel,
        out_type=jax.ShapeDtypeStruct((epad,), F32),
        mesh=mesh,
        scratch_types=[
            pltpu.VMEM((2, 2, CHUNK), jnp.int32),
            pltpu.VMEM((2 * CHUNK, 128), F32),
            pltpu.VMEM((CHUNK,), F32),
            pltpu.SemaphoreType.DMA((2,)),
            pltpu.SemaphoreType.DMA((2,)),
        ],
    )
    def k(dinv_hbm, sdidx_hbm, out_hbm, idx_v, svec, sval_v, isem, gsem):
        cid = lax.axis_index("c")
        sid = lax.axis_index("s")
        wid = cid * 16 + sid
        base_c = wid * per_tile
        zeros16 = jnp.zeros((16,), jnp.int32)
        iota16 = lax.iota(jnp.int32, 16)

        pltpu.async_copy(sdidx_hbm.at[base_c], idx_v.at[0], isem.at[0])

        def body(t, _):
            b = lax.rem(t, 2)
            b1 = 1 - b

            @pl.when(t < per_tile)
            def _prefetch():
                pltpu.make_async_copy(sdidx_hbm.at[base_c + t],
                                      idx_v.at[b], isem.at[b]).wait()
                pltpu.async_copy(dinv_hbm.at[idx_v.at[b, 0]],
                                 svec.at[pl.ds(b * CHUNK, CHUNK)],
                                 gsem.at[b])

            @pl.when(t > 0)
            def _process():
                c = base_c + t - 1
                pltpu.make_async_copy(dinv_hbm.at[idx_v.at[b1, 0]],
                                      svec.at[pl.ds(b1 * CHUNK, CHUNK)],
                                      gsem.at[b1]).wait()
                for g in range(CHUNK // 16):
                    vals = jnp.zeros((16,), F32)
                    for i in range(16):
                        row = svec[b1 * CHUNK + g * 16 + i, 0:16]
                        vals = jnp.where(iota16 == i, row, vals)
                    sval_v[pl.ds(g * 16, 16)] = vals
                pltpu.sync_copy(sval_v, out_hbm.at[pl.ds(c * CHUNK, CHUNK)])

            @pl.when(t + 1 < per_tile)
            def _issue_idx():
                pltpu.async_copy(sdidx_hbm.at[base_c + t + 1],
                                 idx_v.at[b1], isem.at[b1])
            return 0
        lax.fori_loop(0, per_tile + 1, body, 0)

    return k(dinv128, sdidx)


def _edge_pass(hs_tbl, ees_tbl, sdidx, npad):
    d = hs_tbl.shape[1]
    nchunk = sdidx.shape[0]
    per_tile = nchunk // NUM_WORKERS
    rows_per_tile = npad // 16
    nd16 = d // 16
    mesh = plsc.VectorSubcoreMesh(core_axis_name="c", subcore_axis_name="s")

    @functools.partial(
        pl.kernel,
        out_type=jax.ShapeDtypeStruct((2, npad, d), F32),
        mesh=mesh,
        scratch_types=[
            pltpu.VMEM((3, 2, CHUNK), jnp.int32),
            pltpu.VMEM((2, CHUNK, d), F32),
            pltpu.VMEM((2, CHUNK, d), F32),
            pltpu.SemaphoreType.DMA((3,)),
            pltpu.SemaphoreType.DMA((2,)),
            pltpu.SemaphoreType.DMA((2,)),
            pltpu.SemaphoreType.DMA((2,)),
            pltpu.VMEM_SHARED((npad, d), F32),
        ],
    )
    def k(h_hbm, ee_hbm, sdidx_hbm, out_hbm,
          idx_v, hbuf, eebuf, isem, hsem, esem, ssem, acc_sh):
        cid = lax.axis_index("c")
        sid = lax.axis_index("s")
        wid = cid * 16 + sid
        base_c = wid * per_tile
        _zero_rows(hbuf.at[0], nd16)
        _zero_acc_slice(hbuf.at[0], acc_sh, sid, rows_per_tile)
        plsc.subcore_barrier()

        pltpu.async_copy(sdidx_hbm.at[base_c], idx_v.at[0], isem.at[0])

        def body(t, _):
            b = lax.rem(t, 2)
            b1 = 1 - b
            i3 = lax.rem(t, 3)
            i3m1 = lax.rem(t + 2, 3)
            i3m2 = lax.rem(t + 1, 3)

            @pl.when(t >= 2)
            def _drain_scatter():
                pltpu.make_async_copy(hbuf.at[b],
                                      acc_sh.at[idx_v.at[i3m2, 1]],
                                      ssem.at[b]).wait()

            @pl.when(t < per_tile)
            def _issue_gathers():
                c = base_c + t
                pltpu.make_async_copy(sdidx_hbm.at[c], idx_v.at[i3],
                                      isem.at[i3]).wait()
                pltpu.async_copy(h_hbm.at[idx_v.at[i3, 0]], hbuf.at[b],
                                 hsem.at[b])
                pltpu.async_copy(ee_hbm.at[pl.ds(c * CHUNK, CHUNK)],
                                 eebuf.at[b], esem.at[b])

            @pl.when(jnp.logical_and(t > 0, t <= per_tile))
            def _process():
                c = base_c + t - 1
                pltpu.make_async_copy(h_hbm.at[idx_v.at[i3m1, 0]],
                                      hbuf.at[b1], hsem.at[b1]).wait()
                pltpu.make_async_copy(ee_hbm.at[pl.ds(c * CHUNK, CHUNK)],
                                      eebuf.at[b1], esem.at[b1]).wait()

                def edge(e, _):
                    for kk in range(nd16):
                        v = (hbuf[b1, e, pl.ds(kk * 16, 16)]
                             + eebuf[b1, e, pl.ds(kk * 16, 16)])
                        hbuf[b1, e, pl.ds(kk * 16, 16)] = jnp.maximum(v, 0.0)
                    return 0
                lax.fori_loop(0, CHUNK, edge, 0)
                pltpu.async_copy(hbuf.at[b1], acc_sh.at[idx_v.at[i3m1, 1]],
                                 ssem.at[b1], add=True)

            @pl.when(t + 1 < per_tile)
            def _issue_idx():
                pltpu.async_copy(sdidx_hbm.at[base_c + t + 1],
                                 idx_v.at[lax.rem(t + 1, 3)],
                                 isem.at[lax.rem(t + 1, 3)])
            return 0
        lax.fori_loop(0, per_tile + 2, body, 0)
        plsc.subcore_barrier()
        pltpu.sync_copy(acc_sh.at[pl.ds(sid * rows_per_tile, rows_per_tile)],
                        out_hbm.at[cid].at[pl.ds(sid * rows_per_tile, rows_per_tile)])

    return k(hs_tbl, ees_tbl, sdidx)


def _dinv_from_deg(degp):
    npad = degp.shape[1]

    def body(p_ref, o_ref):
        deg = p_ref[0, :, 0:1] + p_ref[1, :, 0:1] + 1.0
        o_ref[:] = jnp.broadcast_to(lax.rsqrt(deg), (npad, 128))
    return pl.pallas_call(
        body,
        out_shape=jax.ShapeDtypeStruct((npad, 128), F32),
    )(degp)


def _node_matmul(xp, wt, b, degp):
    def body(x_ref, w_ref, b_ref, g_ref, o_ref, os_ref):
        deg = g_ref[0, :, 0:1] + g_ref[1, :, 0:1] + 1.0
        dinv = lax.rsqrt(deg)
        h = jnp.dot(x_ref[:], w_ref[:], preferred_element_type=F32) + b_ref[:]
        o_ref[:] = h
        os_ref[:] = h * dinv
    n, dout = xp.shape[0], wt.shape[1]
    return pl.pallas_call(
        body,
        out_shape=[jax.ShapeDtypeStruct((n, dout), F32),
                   jax.ShapeDtypeStruct((n, dout), F32)],
    )(xp, wt, b, degp)


def _edge_matmul(ea8, wt8, b, sval):
    epad = ea8.shape[0]
    dout = wt8.shape[1]
    blk = 4096
    grid = epad // blk

    def body(a_ref, w_ref, b_ref, s_ref, o_ref):
        o = jnp.dot(a_ref[:], w_ref[:], preferred_element_type=F32) + b_ref[:]
        o_ref[:] = o * s_ref[:]
    return pl.pallas_call(
        body,
        grid=(grid,),
        in_specs=[
            pl.BlockSpec((blk, 8), lambda i: (i, 0)),
            pl.BlockSpec((8, dout), lambda i: (0, 0)),
            pl.BlockSpec((1, dout), lambda i: (0, 0)),
            pl.BlockSpec((blk, 1), lambda i: (i, 0)),
        ],
        out_specs=pl.BlockSpec((blk, dout), lambda i: (i, 0)),
        out_shape=jax.ShapeDtypeStruct((epad, dout), F32),
    )(ea8, wt8, b, sval)


def _combine_project(parts, hpre, degp, root, wt, b):
    def body(p_ref, h_ref, g_ref, r_ref, w_ref, b_ref, o_ref, os_ref):
        deg = g_ref[0, :, 0:1] + g_ref[1, :, 0:1] + 1.0
        dinv = lax.rsqrt(deg)
        agg = (p_ref[0] + p_ref[1]) * dinv
        self_t = jnp.maximum(h_ref[:] + r_ref[:], 0.0) / deg
        h = jnp.maximum(agg + self_t, 0.0)
        hcat = jnp.dot(h, w_ref[:], preferred_element_type=F32) + b_ref[:]
        o_ref[:] = hcat
        os_ref[:] = hcat * dinv
    n, dout = hpre.shape[0], wt.shape[1]
    return pl.pallas_call(
        body,
        out_shape=[jax.ShapeDtypeStruct((n, dout), F32),
                   jax.ShapeDtypeStruct((n, dout), F32)],
    )(parts, hpre, degp, root, wt, b)


def _combine_final(parts, hpre, degp, root):
    def body(p_ref, h_ref, g_ref, r_ref, o_ref):
        deg = g_ref[0, :, 0:1] + g_ref[1, :, 0:1] + 1.0
        dinv = lax.rsqrt(deg)
        agg = (p_ref[0] + p_ref[1]) * dinv
        o_ref[:] = agg + jnp.maximum(h_ref[:] + r_ref[:], 0.0) / deg
    return pl.pallas_call(
        body,
        out_shape=jax.ShapeDtypeStruct(hpre.shape, F32),
    )(parts, hpre, degp, root)


def kernel(x, edge_index, edge_attr,
           W1, b1, root1, We1, be1,
           Wmu, bmu, rootmu, Wemu, bemu,
           Wls, bls, rootls, Wels, bels):
    n, d_in = x.shape
    e = edge_index.shape[1]
    d_edge = edge_attr.shape[1]
    npad, epad = _pads(n, e)

    xp = jnp.pad(x, ((0, npad - n), (0, 0)))
    pad_cnt = epad - e
    dump = n + (jnp.arange(pad_cnt, dtype=jnp.int32) % (npad - n))
    sidx = jnp.concatenate([edge_index[0], dump]).reshape(-1, 1, CHUNK)
    didx = jnp.concatenate([edge_index[1], dump]).reshape(-1, 1, CHUNK)
    sdidx = jnp.concatenate([sidx, didx], axis=1)
    ea8 = jnp.pad(edge_attr, ((0, pad_cnt), (0, 8 - d_edge)))

    w1t = W1.T
    wcat_t = jnp.concatenate([Wmu.T, Wls.T], axis=1)
    bcat = jnp.concatenate([bmu, bls]).reshape(1, -1)
    rootcat = jnp.concatenate([rootmu, rootls], axis=1)
    we1t8 = jnp.pad(We1.T, ((0, 8 - d_edge), (0, 0)))
    wecat_t8 = jnp.pad(jnp.concatenate([Wemu.T, Wels.T], axis=1),
                       ((0, 8 - d_edge), (0, 0)))
    becat = jnp.concatenate([bemu, bels]).reshape(1, -1)

    degp = _deg_pass(sdidx, npad)
    dinv128 = _dinv_from_deg(degp)
    sval = _sval_pass(dinv128, sdidx).reshape(-1, 1)

    h1pre, hs1 = _node_matmul(xp, w1t, b1.reshape(1, -1), degp)
    ees1 = _edge_matmul(ea8, we1t8, be1.reshape(1, -1), sval)
    p1 = _edge_pass(hs1, ees1, sdidx, npad)
    hcat, hscat = _combine_project(p1, h1pre, degp, root1, wcat_t, bcat)

    eescat = _edge_matmul(ea8, wecat_t8, becat, sval)
    pcat = _edge_pass(hscat, eescat, sdidx, npad)
    outcat = _combine_final(pcat, hcat, degp, rootcat)

    d_out = Wmu.shape[0]
    return (outcat[:n, :d_out], outcat[:n, d_out:2 * d_out])


# speedup vs baseline: 1.4933x; 1.4933x over previous
import functools
import jax
import jax.numpy as jnp
from jax import lax
from jax.experimental import pallas as pl
from jax.experimental.pallas import tpu as pltpu
from jax.experimental.pallas import tpu_sc as plsc

F32 = jnp.float32
CHUNK = 128
NUM_WORKERS = 32


def _pads(n_nodes, n_edges):
    npad = ((n_nodes + 1 + 127) // 128) * 128
    epad = ((n_edges + NUM_WORKERS * CHUNK - 1) // (NUM_WORKERS * CHUNK)) * (
        NUM_WORKERS * CHUNK)
    return npad, epad


def _zero_rows(buf, ncols16):
    def body(i, _):
        for k in range(ncols16):
            buf[i, pl.ds(k * 16, 16)] = jnp.zeros((16,), F32)
        return 0
    lax.fori_loop(0, CHUNK, body, 0)


def _zero_acc_slice(zbuf, acc_sh, sid, rows_per_tile):
    base = sid * rows_per_tile
    full, rem = rows_per_tile // CHUNK, rows_per_tile % CHUNK
    for r in range(full):
        pltpu.sync_copy(zbuf, acc_sh.at[pl.ds(base + r * CHUNK, CHUNK)])
    if rem:
        pltpu.sync_copy(zbuf.at[pl.ds(0, rem)],
                        acc_sh.at[pl.ds(base + full * CHUNK, rem)])


def _deg_pass(sdidx, npad):
    nchunk = sdidx.shape[0]
    per_tile = nchunk // NUM_WORKERS
    rows_per_tile = npad // 16
    mesh = plsc.VectorSubcoreMesh(core_axis_name="c", subcore_axis_name="s")

    @functools.partial(
        pl.kernel,
        out_type=jax.ShapeDtypeStruct((2, npad, 16), F32),
        mesh=mesh,
        scratch_types=[
            pltpu.VMEM((2, 2, CHUNK), jnp.int32),
            pltpu.VMEM((CHUNK, 16), F32),
            pltpu.VMEM_SHARED((npad, 16), F32),
            pltpu.SemaphoreType.DMA((2,)),
        ],
    )
    def k(sdidx_hbm, out_hbm, idx_v, val_v, acc_sh, isem):
        cid = lax.axis_index("c")
        sid = lax.axis_index("s")
        wid = cid * 16 + sid
        base_g = wid * per_tile
        _zero_rows(val_v, 1)
        _zero_acc_slice(val_v, acc_sh, sid, rows_per_tile)
        plsc.subcore_barrier()

        def fill_ones(i, _):
            val_v[i] = jnp.ones((16,), F32)
            return 0
        lax.fori_loop(0, CHUNK, fill_ones, 0)

        def body(t, _):
            b = lax.rem(t, 2)
            b1 = 1 - b

            @pl.when(t < per_tile)
            def _prefetch():
                pltpu.async_copy(sdidx_hbm.at[base_g + t], idx_v.at[b],
                                 isem.at[b])

            @pl.when(t > 0)
            def _process():
                pltpu.make_async_copy(sdidx_hbm.at[base_g + t - 1],
                                      idx_v.at[b1], isem.at[b1]).wait()
                pltpu.sync_copy(val_v, acc_sh.at[idx_v.at[b1, 0]], add=True)
            return 0
        lax.fori_loop(0, per_tile + 1, body, 0)
        plsc.subcore_barrier()
        pltpu.sync_copy(acc_sh.at[pl.ds(sid * rows_per_tile, rows_per_tile)],
                        out_hbm.at[cid].at[pl.ds(sid * rows_per_tile, rows_per_tile)])

    return k(sdidx)


def _edge_pass(h_tbl, ee_tbl, dinv128, sdidx, npad):
    d = h_tbl.shape[1]
    nchunk = sdidx.shape[0]
    per_tile = nchunk // NUM_WORKERS
    rows_per_tile = npad // 16
    nd16 = d // 16
    mesh = plsc.VectorSubcoreMesh(core_axis_name="c", subcore_axis_name="s")

    @functools.partial(
        pl.kernel,
        out_type=jax.ShapeDtypeStruct((2, npad, d), F32),
        mesh=mesh,
        scratch_types=[
            pltpu.VMEM((2, CHUNK), jnp.int32),
            pltpu.VMEM((CHUNK, d), F32),
            pltpu.VMEM((CHUNK, d), F32),
            pltpu.VMEM((CHUNK, 128), F32),
            pltpu.SemaphoreType.DMA,
            pltpu.SemaphoreType.DMA,
            pltpu.SemaphoreType.DMA,
            pltpu.VMEM_SHARED((npad, d), F32),
        ],
    )
    def k(h_hbm, ee_hbm, dinv_hbm, sdidx_hbm, out_hbm,
          idx_v, hbuf, eebuf, svec, hsem, esem, vsem, acc_sh):
        cid = lax.axis_index("c")
        sid = lax.axis_index("s")
        wid = cid * 16 + sid
        base_c = wid * per_tile
        _zero_rows(hbuf, nd16)
        _zero_acc_slice(hbuf, acc_sh, sid, rows_per_tile)
        plsc.subcore_barrier()

        def body(t, _):
            c = base_c + t
            pltpu.sync_copy(sdidx_hbm.at[c], idx_v)
            pltpu.async_copy(h_hbm.at[idx_v.at[0]], hbuf, hsem)
            pltpu.async_copy(dinv_hbm.at[idx_v.at[0]], svec, vsem)
            pltpu.async_copy(ee_hbm.at[pl.ds(c * CHUNK, CHUNK)], eebuf, esem)
            pltpu.make_async_copy(h_hbm.at[idx_v.at[0]], hbuf, hsem).wait()
            pltpu.make_async_copy(dinv_hbm.at[idx_v.at[0]], svec, vsem).wait()
            pltpu.make_async_copy(ee_hbm.at[pl.ds(c * CHUNK, CHUNK)],
                                  eebuf, esem).wait()

            def edge(e, _):
                s = svec[e, 0:16]
                for kk in range(nd16):
                    v = (hbuf[e, pl.ds(kk * 16, 16)]
                         + eebuf[e, pl.ds(kk * 16, 16)])
                    hbuf[e, pl.ds(kk * 16, 16)] = jnp.maximum(v, 0.0) * s
                return 0
            lax.fori_loop(0, CHUNK, edge, 0)
            pltpu.sync_copy(hbuf, acc_sh.at[idx_v.at[1]], add=True)
            return 0
        lax.fori_loop(0, per_tile, body, 0)
        plsc.subcore_barrier()
        pltpu.sync_copy(acc_sh.at[pl.ds(sid * rows_per_tile, rows_per_tile)],
                        out_hbm.at[cid].at[pl.ds(sid * rows_per_tile, rows_per_tile)])

    return k(h_tbl, ee_tbl, dinv128, sdidx)


def _dinv_from_deg(degp):
    npad = degp.shape[1]

    def body(p_ref, o_ref):
        deg = p_ref[0, :, 0:1] + p_ref[1, :, 0:1] + 1.0
        o_ref[:] = jnp.broadcast_to(lax.rsqrt(deg), (npad, 128))
    return pl.pallas_call(
        body,
        out_shape=jax.ShapeDtypeStruct((npad, 128), F32),
    )(degp)


def _node_matmul(xp, wt, b):
    def body(x_ref, w_ref, b_ref, o_ref):
        o_ref[:] = jnp.dot(x_ref[:], w_ref[:],
                           preferred_element_type=F32) + b_ref[:]
    return pl.pallas_call(
        body,
        out_shape=jax.ShapeDtypeStruct((xp.shape[0], wt.shape[1]), F32),
    )(xp, wt, b)


def _edge_matmul(ea8, wt8, b):
    epad = ea8.shape[0]
    dout = wt8.shape[1]
    blk = 4096
    grid = epad // blk

    def body(a_ref, w_ref, b_ref, o_ref):
        o_ref[:] = jnp.dot(a_ref[:], w_ref[:],
                           preferred_element_type=F32) + b_ref[:]
    return pl.pallas_call(
        body,
        grid=(grid,),
        in_specs=[
            pl.BlockSpec((blk, 8), lambda i: (i, 0)),
            pl.BlockSpec((8, dout), lambda i: (0, 0)),
            pl.BlockSpec((1, dout), lambda i: (0, 0)),
        ],
        out_specs=pl.BlockSpec((blk, dout), lambda i: (i, 0)),
        out_shape=jax.ShapeDtypeStruct((epad, dout), F32),
    )(ea8, wt8, b)


def _combine_project(parts, hpre, degp, root, wt, b):
    def body(p_ref, h_ref, g_ref, r_ref, w_ref, b_ref, o_ref):
        deg = g_ref[0, :, 0:1] + g_ref[1, :, 0:1] + 1.0
        dinv = lax.rsqrt(deg)
        agg = (p_ref[0] + p_ref[1]) * dinv
        self_t = jnp.maximum(h_ref[:] + r_ref[:], 0.0) / deg
        h = jnp.maximum(agg + self_t, 0.0)
        o_ref[:] = jnp.dot(h, w_ref[:], preferred_element_type=F32) + b_ref[:]
    return pl.pallas_call(
        body,
        out_shape=jax.ShapeDtypeStruct((hpre.shape[0], wt.shape[1]), F32),
    )(parts, hpre, degp, root, wt, b)


def _combine_final(parts, hpre, degp, root):
    def body(p_ref, h_ref, g_ref, r_ref, o_ref):
        deg = g_ref[0, :, 0:1] + g_ref[1, :, 0:1] + 1.0
        dinv = lax.rsqrt(deg)
        agg = (p_ref[0] + p_ref[1]) * dinv
        o_ref[:] = agg + jnp.maximum(h_ref[:] + r_ref[:], 0.0) / deg
    return pl.pallas_call(
        body,
        out_shape=jax.ShapeDtypeStruct(hpre.shape, F32),
    )(parts, hpre, degp, root)


def kernel(x, edge_index, edge_attr,
           W1, b1, root1, We1, be1,
           Wmu, bmu, rootmu, Wemu, bemu,
           Wls, bls, rootls, Wels, bels):
    n, d_in = x.shape
    e = edge_index.shape[1]
    d_edge = edge_attr.shape[1]
    npad, epad = _pads(n, e)

    xp = jnp.pad(x, ((0, npad - n), (0, 0)))
    pad_cnt = epad - e
    dump = n + (jnp.arange(pad_cnt, dtype=jnp.int32) % (npad - n))
    sidx = jnp.concatenate([edge_index[0], dump]).reshape(-1, 1, CHUNK)
    didx = jnp.concatenate([edge_index[1], dump]).reshape(-1, 1, CHUNK)
    sdidx = jnp.concatenate([sidx, didx], axis=1)
    ea8 = jnp.pad(edge_attr, ((0, pad_cnt), (0, 8 - d_edge)))

    w1t = W1.T
    wcat_t = jnp.concatenate([Wmu.T, Wls.T], axis=1)
    bcat = jnp.concatenate([bmu, bls]).reshape(1, -1)
    rootcat = jnp.concatenate([rootmu, rootls], axis=1)
    we1t8 = jnp.pad(We1.T, ((0, 8 - d_edge), (0, 0)))
    wecat_t8 = jnp.pad(jnp.concatenate([Wemu.T, Wels.T], axis=1),
                       ((0, 8 - d_edge), (0, 0)))
    becat = jnp.concatenate([bemu, bels]).reshape(1, -1)

    degp = _deg_pass(sdidx, npad)
    dinv128 = _dinv_from_deg(degp)

    h1pre = _node_matmul(xp, w1t, b1.reshape(1, -1))
    ee1 = _edge_matmul(ea8, we1t8, be1.reshape(1, -1))
    p1 = _edge_pass(h1pre, ee1, dinv128, sdidx, npad)
    hcat = _combine_project(p1, h1pre, degp, root1, wcat_t, bcat)

    eecat = _edge_matmul(ea8, wecat_t8, becat)
    pcat = _edge_pass(hcat, eecat, dinv128, sdidx, npad)
    outcat = _combine_final(pcat, hcat, degp, rootcat)

    d_out = Wmu.shape[0]
    return (outcat[:n, :d_out], outcat[:n, d_out:2 * d_out])
